# baseline (speedup 1.0000x reference)
import jax, jax.numpy as jnp
from jax.experimental import pallas as pl


def _fps(xyz, npoint):
    xyz = jax.lax.stop_gradient(xyz)
    B, N, _ = xyz.shape
    def step(carry, _):
        dists, farthest = carry
        centroid = jnp.take_along_axis(xyz, farthest[:, None, None], axis=1)
        d = jnp.sum((xyz - centroid) ** 2, axis=-1)
        dists = jnp.minimum(dists, d)
        nxt = jnp.argmax(dists, axis=-1).astype(jnp.int32)
        return (dists, nxt), farthest
    _, idxs = jax.lax.scan(step, (jnp.full((B, N), 1e10, xyz.dtype), jnp.zeros((B,), jnp.int32)), None, length=npoint)
    return jnp.transpose(idxs)


def _index_points(points, idx):
    B = points.shape[0]
    flat = idx.reshape(B, -1)
    out = jnp.take_along_axis(points, flat[..., None], axis=1)
    return out.reshape(idx.shape + (points.shape[-1],))


def _ball_query(radius, nsample, xyz, new_xyz):
    B, N, _ = xyz.shape
    S = new_xyz.shape[1]
    d = jax.lax.stop_gradient(jnp.sum((new_xyz[:, :, None, :] - xyz[:, None, :, :]) ** 2, axis=-1))
    gidx = jnp.broadcast_to(jnp.arange(N, dtype=jnp.int32), (B, S, N))
    gidx = jnp.where(d > radius * radius, N, gidx)
    gidx = jnp.sort(gidx, axis=-1)[:, :, :nsample]
    first = gidx[:, :, 0:1]
    gidx = jnp.where(gidx == N, jnp.broadcast_to(first, gidx.shape), gidx)
    return jnp.clip(gidx, 0, N - 1)


def _bn(x, gamma, beta, axes):
    mean = jnp.mean(x, axis=axes, keepdims=True)
    var = jnp.var(x, axis=axes, keepdims=True)
    return gamma * (x - mean) / jnp.sqrt(var + 1e-5) + beta


def _sa_pool(xyz, feats, npoint, radius, nsample, W, gamma, beta):
    fidx = _fps(xyz, npoint)
    new_xyz = _index_points(xyz, fidx)
    idx = _ball_query(radius, nsample, xyz, new_xyz)
    grouped = _index_points(xyz, idx) - new_xyz[:, :, None, :]
    if feats is not None:
        grouped = jnp.concatenate([grouped, _index_points(feats, idx)], axis=-1)
    h = jnp.einsum('bskc,oc->bsko', grouped, W)
    h = jax.nn.relu(_bn(h, gamma, beta, (0, 1, 2)))
    return new_xyz, jnp.max(h, axis=2)


def _sa_enhanced(xyz, feats, npoint, radius, nsample, Wphi, gamma, beta, Wpsi, bpsi):
    B = xyz.shape[0]
    fidx = _fps(xyz, npoint)
    new_xyz = _index_points(xyz, fidx)
    new_feats = _index_points(feats, fidx)
    idx = _ball_query(radius, nsample, xyz, new_xyz)
    grouped_xyz = _index_points(xyz, idx) - new_xyz[:, :, None, :]
    grouped = jnp.concatenate([grouped_xyz, _index_points(feats, idx)], axis=-1)
    G, O, I = Wphi.shape
    S, K = grouped.shape[1], grouped.shape[2]
    x = grouped.reshape(B, S, K, G, I)
    h = jnp.einsum('bskgi,goi->bskgo', x, Wphi).reshape(B, S, K, G * O)
    h = jax.nn.relu(_bn(h, gamma, beta, (0, 1, 2)))
    pooled = jnp.max(h, axis=2)
    new = jax.nn.relu(jnp.einsum('bsc,oc->bso', pooled, Wpsi) + bpsi)
    return new_xyz, jnp.concatenate([new_feats, new], axis=-1)


def _identity_pallas(x):
    def body(x_ref, o_ref):
        o_ref[...] = x_ref[...]
    return pl.pallas_call(body, out_shape=jax.ShapeDtypeStruct(x.shape, x.dtype))(x)


def kernel(pointcloud, W1, g1, b1, W2, g2, b2, Wphi3, g3, b3, Wpsi3, bp3, Wphi4, g4, b4, Wpsi4, bp4, Wphi5, g5, b5, Wpsi5, bp5, W6, g6, b6, Wf1, gf1, bf1, Wf2, gf2, bf2, Wf3, bf3):
    xyz = pointcloud[..., 0:3]
    xyz, f = _sa_pool(xyz, None, 512, 0.25, 64, W1, g1, b1)
    xyz, f = _sa_pool(xyz, f, 128, 0.32, 64, W2, g2, b2)
    xyz, f = _sa_enhanced(xyz, f, 128, 0.39, 16, Wphi3, g3, b3, Wpsi3, bp3)
    xyz, f = _sa_enhanced(xyz, f, 128, 0.39, 16, Wphi4, g4, b4, Wpsi4, bp4)
    xyz, f = _sa_enhanced(xyz, f, 128, 0.39, 16, Wphi5, g5, b5, Wpsi5, bp5)
    grouped = jnp.concatenate([xyz, f], axis=-1)
    h = jax.nn.relu(_bn(jnp.einsum('bnc,oc->bno', grouped, W6), g6, b6, (0, 1)))
    gfeat = jnp.max(h, axis=1)
    h = jax.nn.relu(_bn(gfeat @ Wf1.T, gf1, bf1, (0,)))
    h = jax.nn.relu(_bn(h @ Wf2.T, gf2, bf2, (0,)))
    out = h @ Wf3.T + bf3
    return _identity_pallas(out)


# speedup vs baseline: 1.9599x; 1.9599x over previous
import functools

import jax
import jax.numpy as jnp
import numpy as np
from jax.experimental import pallas as pl
from jax.experimental.pallas import tpu as pltpu

_pcall = pl.pallas_call
_HI = jax.lax.Precision.HIGHEST
_NEG = np.float32(-3.0e38)


def _dot(a, b, ca, cb):
    return jax.lax.dot_general(a, b, (((ca,), (cb,)), ((), ())), precision=_HI,
                               preferred_element_type=jnp.float32)


def _bdot(a, b, ca, cb):
    return jax.lax.dot_general(a.astype(jnp.bfloat16), b.astype(jnp.bfloat16),
                               (((ca,), (cb,)), ((), ())),
                               preferred_element_type=jnp.float32)


def _fps_body(npoint, xs_ref, out_ref, d_ref, f_ref):
    X = xs_ref[0]
    Y = xs_ref[1]
    Z = xs_ref[2]
    B, N = X.shape
    li = jax.lax.broadcasted_iota(jnp.int32, (B, N), 1)
    lsel = jax.lax.broadcasted_iota(jnp.int32, (B, npoint), 1)
    d_ref[...] = jnp.full((B, N), 1e10, jnp.float32)
    f_ref[...] = jnp.zeros((B, 128), jnp.int32)
    out_ref[...] = jnp.zeros((B, npoint), jnp.int32)

    def step(t, _):
        f = f_ref[:, 0:1]
        cur = out_ref[...]
        msk = (lsel == t).astype(jnp.int32)
        out_ref[...] = cur + msk * (jnp.broadcast_to(f, cur.shape) - cur)
        fb = jnp.broadcast_to(f, (B, N))
        cx = jnp.sum(jnp.where(li == fb, X, 0.0), axis=1, keepdims=True)
        cy = jnp.sum(jnp.where(li == fb, Y, 0.0), axis=1, keepdims=True)
        cz = jnp.sum(jnp.where(li == fb, Z, 0.0), axis=1, keepdims=True)
        dx = X - cx
        dy = Y - cy
        dz = Z - cz
        d = (dx * dx + dy * dy) + dz * dz
        dists = jnp.minimum(d_ref[...], d)
        d_ref[...] = dists
        m = jnp.max(dists, axis=1, keepdims=True)
        f = jnp.min(jnp.where(dists == m, li, N), axis=1, keepdims=True)
        f_ref[...] = jnp.broadcast_to(f, (B, 128))
        return 0

    jax.lax.fori_loop(0, npoint, step, 0)


def _fps_call(xyz, npoint):
    B, N, _ = xyz.shape
    xs = jnp.transpose(xyz, (2, 0, 1))
    return _pcall(
        functools.partial(_fps_body, npoint),
        out_shape=jax.ShapeDtypeStruct((B, npoint), jnp.int32),
        scratch_shapes=[pltpu.VMEM((B, N), jnp.float32),
                        pltpu.VMEM((B, 128), jnp.int32)],
    )(xs)


def _sa_body(N, S, K, C, O, F, thr, p_ref, xs_ref, fidx_ref, fidxr_ref, w_ref,
             wrep_ref, maxh_ref, nxyz_ref, ssum_ref, ssq_ref, *rest):
    if F:
        nfeat_ref = rest[0]
        csum_ref, csq_ref = rest[1], rest[2]
    else:
        csum_ref, csq_ref = rest[0], rest[1]
    b = pl.program_id(0)
    Pb = p_ref[0]
    xs3 = xs_ref[0]
    fcol = fidx_ref[0]
    frow = fidxr_ref[0]
    W = w_ref[...]

    OH = (fcol == jax.lax.broadcasted_iota(jnp.int32, (S, N), 1)).astype(jnp.float32)
    OHT = (frow == jax.lax.broadcasted_iota(jnp.int32, (N, S), 0)).astype(jnp.float32)
    xr = xs3[0:1, :]
    yr = xs3[1:2, :]
    zr = xs3[2:3, :]
    nxx = jnp.sum(OH * xr, axis=1, keepdims=True)
    nxy = jnp.sum(OH * yr, axis=1, keepdims=True)
    nxz = jnp.sum(OH * zr, axis=1, keepdims=True)
    nxyz = jnp.concatenate([nxx, nxy, nxz], axis=1)
    cxr = jnp.sum(OHT * Pb[:, 0:1], axis=0, keepdims=True)
    cyr = jnp.sum(OHT * Pb[:, 1:2], axis=0, keepdims=True)
    czr = jnp.sum(OHT * Pb[:, 2:3], axis=0, keepdims=True)
    nxyz_ref[0] = nxyz

    dx = cxr - Pb[:, 0:1]
    dy = cyr - Pb[:, 1:2]
    dz = czr - Pb[:, 2:3]
    DT = (dx * dx + dy * dy) + dz * dz
    maskT = jnp.logical_not(DT > thr)

    mbf = maskT.astype(jnp.bfloat16)
    io0 = jax.lax.broadcasted_iota(jnp.int32, (N, N), 0)
    io1 = jax.lax.broadcasted_iota(jnp.int32, (N, N), 1)
    LT = (io1 <= io0).astype(jnp.bfloat16)
    RANKT = jax.lax.dot_general(LT, mbf, (((1,), (0,)), ((), ())),
                                preferred_element_type=jnp.float32)

    SELT = jnp.where(maskT & (RANKT <= K), 1.0, 0.0).astype(jnp.float32)
    FIRSTT = jnp.where(maskT & (RANKT == 1.0), 1.0, 0.0).astype(jnp.float32)

    @pl.when(b == 0)
    def _():
        ssum_ref[...] = jnp.zeros_like(ssum_ref)
        ssq_ref[...] = jnp.zeros_like(ssq_ref)
        csum_ref[...] = jnp.zeros_like(csum_ref)
        csq_ref[...] = jnp.zeros_like(csq_ref)

    xyz3 = Pb[:, 0:3]
    wrepb = wrep_ref[...]
    if C > 3:
        hfeat = jax.lax.dot_general(
            Pb[:, 3:C].astype(jnp.bfloat16), W[:, 3:C].astype(jnp.bfloat16),
            (((1,), (1,)), ((), ())), preferred_element_type=jnp.float32)

    def blk(i, _):
        s0 = i * 8
        E8 = ((jax.lax.broadcasted_iota(jnp.int32, (S, 8), 0) - s0)
              == jax.lax.broadcasted_iota(jnp.int32, (S, 8), 1)).astype(jnp.float32)
        cols = _dot(SELT, E8, 1, 0)
        fcols = _dot(FIRSTT, E8, 1, 0)
        parts = []
        for r in range(8):
            c1 = nxyz_ref[0, pl.ds(s0 + r, 1), :]
            parts.append(xyz3 - c1)
        D24 = jnp.concatenate(parts, axis=1).astype(jnp.bfloat16)
        hs8 = jax.lax.dot_general(D24, wrepb, (((1,), (0,)), ((), ())),
                                  preferred_element_type=jnp.float32)
        bsum = jnp.zeros((1, O), jnp.float32)
        bsq = jnp.zeros((1, O), jnp.float32)
        for r in range(8):
            hsr = hs8[:, 128 * r:128 * r + O]
            if C > 3:
                hsr = hsr + hfeat
            colr = cols[:, r:r + 1]
            msk = colr > 0.5
            red = jnp.max(jnp.where(msk, hsr, _NEG), axis=0, keepdims=True)
            maxh_ref[0, pl.ds(s0 + r, 1), :] = red
            hz = jnp.where(msk, hsr, 0.0)
            hfirst = jnp.sum(jnp.where(fcols[:, r:r + 1] > 0.5, hsr, 0.0),
                             axis=0, keepdims=True)
            pad = K - jnp.sum(colr)
            bsum = bsum + (jnp.sum(hz, axis=0, keepdims=True) + pad * hfirst)
            bsq = bsq + (jnp.sum(hz * hz, axis=0, keepdims=True)
                         + pad * (hfirst * hfirst))
        y = bsum - csum_ref[...]
        t = ssum_ref[...] + y
        csum_ref[...] = (t - ssum_ref[...]) - y
        ssum_ref[...] = t
        y2 = bsq - csq_ref[...]
        t2 = ssq_ref[...] + y2
        csq_ref[...] = (t2 - ssq_ref[...]) - y2
        ssq_ref[...] = t2
        return 0

    jax.lax.fori_loop(0, S // 8, blk, 0)

    if F:
        nfeat_ref[0] = _dot(OH, Pb[:, 3:3 + F], 1, 0)


def _sa_call(P, fidx, W, K, thr, feats_out):
    B, N, C = P.shape
    S = fidx.shape[1]
    O = W.shape[0]
    F = C - 3 if feats_out else 0
    xs = jnp.transpose(P[..., 0:3], (0, 2, 1))
    fidxT = fidx[..., None]
    fidxR = fidx[:, None, :]
    w3t = W[:, 0:3].T
    wrep = jnp.zeros((24, 8 * 128), jnp.float32)
    for r in range(8):
        wrep = wrep.at[3 * r:3 * r + 3, 128 * r:128 * r + O].set(w3t)
    wrep = wrep.astype(jnp.bfloat16)
    out_shape = [
        jax.ShapeDtypeStruct((B, S, O), jnp.float32),
        jax.ShapeDtypeStruct((B, S, 3), jnp.float32),
        jax.ShapeDtypeStruct((1, O), jnp.float32),
        jax.ShapeDtypeStruct((1, O), jnp.float32),
    ]
    out_specs = [
        pl.BlockSpec((1, S, O), lambda b: (b, 0, 0)),
        pl.BlockSpec((1, S, 3), lambda b: (b, 0, 0)),
        pl.BlockSpec((1, O), lambda b: (0, 0)),
        pl.BlockSpec((1, O), lambda b: (0, 0)),
    ]
    if feats_out:
        out_shape.append(jax.ShapeDtypeStruct((B, S, F), jnp.float32))
        out_specs.append(pl.BlockSpec((1, S, F), lambda b: (b, 0, 0)))
    return _pcall(
        functools.partial(_sa_body, N, S, K, C, O, F, thr),
        grid=(B,),
        in_specs=[
            pl.BlockSpec((1, N, C), lambda b: (b, 0, 0)),
            pl.BlockSpec((1, 3, N), lambda b: (b, 0, 0)),
            pl.BlockSpec((1, S, 1), lambda b: (b, 0, 0)),
            pl.BlockSpec((1, 1, S), lambda b: (b, 0, 0)),
            pl.BlockSpec((O, C), lambda b: (0, 0)),
            pl.BlockSpec((24, 8 * 128), lambda b: (0, 0)),
        ],
        out_specs=out_specs,
        out_shape=out_shape,
        scratch_shapes=[pltpu.VMEM((1, O), jnp.float32),
                        pltpu.VMEM((1, O), jnp.float32)],
    )(P, xs, fidxT, fidxR, W, wrep)


def _fin_body(cnt, psi, maxh_ref, ssum_ref, ssq_ref, g_ref, b_ref, *rest):
    mean = ssum_ref[...] / cnt
    var = ssq_ref[...] / cnt - mean * mean
    x = maxh_ref[0]
    y = g_ref[...] * (x - mean) / jnp.sqrt(var + 1e-5) + b_ref[...]
    y = jnp.maximum(y, 0.0)
    if psi:
        wpsi_ref, bp_ref, pooled_ref, new_ref = rest
        pooled_ref[0] = y
        z = _bdot(y, wpsi_ref[...], 1, 1) + bp_ref[...]
        new_ref[0] = jnp.maximum(z, 0.0)
    else:
        rest[0][0] = y


def _fin_call(maxh, ssum, ssq, g, bb, K, wpsi=None, bp=None):
    B, S, O = maxh.shape
    cnt = np.float32(B * S * K)
    psi = wpsi is not None
    ins = [maxh, ssum, ssq, g.reshape(1, O), bb.reshape(1, O)]
    in_specs = [
        pl.BlockSpec((1, S, O), lambda b: (b, 0, 0)),
        pl.BlockSpec((1, O), lambda b: (0, 0)),
        pl.BlockSpec((1, O), lambda b: (0, 0)),
        pl.BlockSpec((1, O), lambda b: (0, 0)),
        pl.BlockSpec((1, O), lambda b: (0, 0)),
    ]
    if psi:
        P = wpsi.shape[0]
        ins += [wpsi, bp.reshape(1, P)]
        in_specs += [pl.BlockSpec(wpsi.shape, lambda b: (0, 0)),
                     pl.BlockSpec((1, P), lambda b: (0, 0))]
        out_shape = [jax.ShapeDtypeStruct((B, S, O), jnp.float32),
                     jax.ShapeDtypeStruct((B, S, P), jnp.float32)]
        out_specs = [pl.BlockSpec((1, S, O), lambda b: (b, 0, 0)),
                     pl.BlockSpec((1, S, P), lambda b: (b, 0, 0))]
    else:
        out_shape = [jax.ShapeDtypeStruct((B, S, O), jnp.float32)]
        out_specs = [pl.BlockSpec((1, S, O), lambda b: (b, 0, 0))]
    res = _pcall(
        functools.partial(_fin_body, cnt, psi),
        grid=(B,),
        in_specs=in_specs,
        out_specs=out_specs,
        out_shape=out_shape,
    )(*ins)
    return res[1] if psi else res[0]


def _bn_rows(h, g, bb):
    mean = jnp.mean(h, axis=0, keepdims=True)
    var = jnp.mean((h - mean) ** 2, axis=0, keepdims=True)
    return g * (h - mean) / jnp.sqrt(var + 1e-5) + bb


def _head_body(B, NPB, x_ref, w6_ref, g6_ref, b6_ref, wf1_ref, gf1_ref,
               bf1_ref, wf2_ref, gf2_ref, bf2_ref, wf3_ref, bf3_ref, out_ref):
    X = x_ref[...]
    h = _bdot(X, w6_ref[...], 1, 1)
    h = jnp.maximum(_bn_rows(h, g6_ref[...], b6_ref[...]), 0.0)
    rows = [jnp.max(h[i * NPB:(i + 1) * NPB, :], axis=0, keepdims=True)
            for i in range(B)]
    gf = jnp.concatenate(rows, axis=0)
    h1 = _bdot(gf, wf1_ref[...], 1, 1)
    h1 = jnp.maximum(_bn_rows(h1, gf1_ref[...], bf1_ref[...]), 0.0)
    h2 = _bdot(h1, wf2_ref[...], 1, 1)
    h2 = jnp.maximum(_bn_rows(h2, gf2_ref[...], bf2_ref[...]), 0.0)
    out_ref[...] = _bdot(h2, wf3_ref[...], 1, 1) + bf3_ref[...]


def _head_call(X, B, W6, g6, b6, Wf1, gf1, bf1, Wf2, gf2, bf2, Wf3, bf3):
    NPB = X.shape[0] // B
    return _pcall(
        functools.partial(_head_body, B, NPB),
        out_shape=jax.ShapeDtypeStruct((B, Wf3.shape[0]), jnp.float32),
    )(X, W6, g6.reshape(1, -1), b6.reshape(1, -1), Wf1, gf1.reshape(1, -1),
      bf1.reshape(1, -1), Wf2, gf2.reshape(1, -1), bf2.reshape(1, -1),
      Wf3, bf3.reshape(1, -1))


def _blockdiag(Wphi):
    G, O, I = Wphi.shape
    out = jnp.zeros((G * O, G * I), jnp.float32)
    for g in range(G):
        out = out.at[g * O:(g + 1) * O, g * I:(g + 1) * I].set(Wphi[g])
    return out


def kernel(pointcloud, W1, g1, b1, W2, g2, b2, Wphi3, g3, b3, Wpsi3, bp3,
           Wphi4, g4, b4, Wpsi4, bp4, Wphi5, g5, b5, Wpsi5, bp5, W6, g6, b6,
           Wf1, gf1, bf1, Wf2, gf2, bf2, Wf3, bf3):
    B = pointcloud.shape[0]
    xyz = pointcloud[..., 0:3]

    fidx = _fps_call(xyz, 512)
    maxh, nxyz, ssum, ssq = _sa_call(xyz, fidx, W1, 64,
                                     np.float32(0.25 * 0.25), False)
    f = _fin_call(maxh, ssum, ssq, g1, b1, 64)

    fidx = _fps_call(nxyz, 128)
    P = jnp.concatenate([nxyz, f], axis=-1)
    maxh, nxyz, ssum, ssq = _sa_call(P, fidx, W2, 64,
                                     np.float32(0.32 * 0.32), False)
    f = _fin_call(maxh, ssum, ssq, g2, b2, 64)

    for Wphi, g, bb, Wpsi, bp in ((Wphi3, g3, b3, Wpsi3, bp3),
                                  (Wphi4, g4, b4, Wpsi4, bp4),
                                  (Wphi5, g5, b5, Wpsi5, bp5)):
        fidx = _fps_call(nxyz, 128)
        P = jnp.concatenate([nxyz, f], axis=-1)
        Wbd = _blockdiag(Wphi)
        maxh, nxyz2, ssum, ssq, nfeat = _sa_call(P, fidx, Wbd, 16,
                                                 np.float32(0.39 * 0.39), True)
        new = _fin_call(maxh, ssum, ssq, g, bb, 16, Wpsi, bp)
        f = jnp.concatenate([nfeat, new], axis=-1)
        nxyz = nxyz2

    X = jnp.concatenate([nxyz, f], axis=-1)
    X = X.reshape(B * X.shape[1], X.shape[2])
    return _head_call(X, B, W6, g6, b6, Wf1, gf1, bf1, Wf2, gf2, bf2, Wf3, bf3)
